# Initial kernel scaffold; baseline (speedup 1.0000x reference)
import jax
import jax.numpy as jnp
from jax.experimental import pallas as pl


def kernel(x_topo, edge_index_topo, x_topo_batch, x_lc, edge_index_lc, x_lc_batch, params):
    raise NotImplementedError("write your pallas kernel here")


# speedup vs baseline: 1.0448x; 1.0448x over previous
import functools

import jax
import jax.numpy as jnp
from jax.experimental import pallas as pl
from jax.experimental.pallas import tpu as pltpu

NB = 50
KTOP = 30
KPAD = 32
HIDS = [128, 128]
DOUT = 64


def _leaky(x, s=0.01):
    return jnp.where(x >= 0, x, s * x)


def _topk_body(keys_ref, batch_ref, idx_ref, ok_ref, valid_ref):
    b = pl.program_id(0)
    keys = keys_ref[...]
    bat = batch_ref[...]
    rows = jax.lax.broadcasted_iota(jnp.int32, keys.shape, 0)
    lanes = jax.lax.broadcasted_iota(jnp.int32, keys.shape, 1)
    lin = rows * 128 + lanes
    neg_inf = jnp.float32(-jnp.inf)
    big = jnp.int32(2**30)
    kiota = jax.lax.broadcasted_iota(jnp.int32, (1, KPAD), 1)

    valid_ref[...] = (bat == b).astype(jnp.int32)
    idx_ref[0, :, :] = jnp.zeros((1, KPAD), jnp.int32)
    ok_ref[0, :, :] = jnp.zeros((1, KPAD), jnp.int32)

    def body(k, carry):
        valid = valid_ref[...] != 0
        mk = jnp.where(valid, keys, neg_inf)
        m = jnp.max(mk)
        has = m > neg_inf
        cand = jnp.where(valid & (keys == m), lin, big)
        idx = jnp.min(cand)
        sel = (kiota == k) & has
        idx_ref[0, :, :] = jnp.where(sel, idx, idx_ref[0, :, :])
        ok_ref[0, :, :] = jnp.where(sel, 1, ok_ref[0, :, :])
        valid_ref[...] = jnp.where(lin != idx, valid_ref[...], 0)
        return carry

    jax.lax.fori_loop(0, KTOP, body, 0)


def _sort_pool(x, batch, k):
    n, c = x.shape
    npad = ((n + 1023) // 1024) * 1024
    keys = x[:, -1]
    keys_p = jnp.full((npad,), -jnp.inf, jnp.float32).at[:n].set(keys)
    batch_p = jnp.full((npad,), -1, jnp.int32).at[:n].set(batch.astype(jnp.int32))
    r = npad // 128
    keys2 = keys_p.reshape(r, 128)
    batch2 = batch_p.reshape(r, 128)

    idx, ok = pl.pallas_call(
        _topk_body,
        grid=(NB,),
        in_specs=[
            pl.BlockSpec((r, 128), lambda b: (0, 0)),
            pl.BlockSpec((r, 128), lambda b: (0, 0)),
        ],
        out_specs=[
            pl.BlockSpec((1, 1, KPAD), lambda b: (b, 0, 0)),
            pl.BlockSpec((1, 1, KPAD), lambda b: (b, 0, 0)),
        ],
        out_shape=[
            jax.ShapeDtypeStruct((NB, 1, KPAD), jnp.int32),
            jax.ShapeDtypeStruct((NB, 1, KPAD), jnp.int32),
        ],
        scratch_shapes=[pltpu.VMEM((r, 128), jnp.int32)],
    )(keys2, batch2)

    idx = idx.reshape(NB, KPAD)[:, :k].reshape(-1)
    ok = ok.reshape(NB, KPAD)[:, :k].reshape(-1)
    rows = x[idx] * ok[:, None].astype(x.dtype)
    return rows.reshape(NB, k * c)


def _gat(x, ei, W, a_s, a_d, bb):
    n = x.shape[0]
    src = ei[0].astype(jnp.int32)
    dst = ei[1].astype(jnp.int32)
    loop = jnp.arange(n, dtype=jnp.int32)
    seg = jnp.where(src == dst, jnp.int32(n), dst)
    src = jnp.concatenate([src, loop])
    dst = jnp.concatenate([dst, loop])
    seg = jnp.concatenate([seg, loop])
    xl = x @ W
    al = (xl * a_s).sum(-1)
    ad = (xl * a_d).sum(-1)
    e = al[src] + ad[dst]
    e = jnp.where(e >= 0, e, 0.2 * e)
    emax = jax.ops.segment_max(e, seg, num_segments=n + 1)
    ex = jnp.exp(e - emax[seg])
    den = jax.ops.segment_sum(ex, seg, num_segments=n + 1)
    alpha = ex / (den[seg] + 1e-16)
    out = jax.ops.segment_sum(alpha[:, None] * xl[src], seg, num_segments=n + 1)[:n]
    return out + bb


def _gnn(x, ei, batch, p, g):
    for i in range(len(HIDS)):
        x = _gat(x, ei, p[g + "_gatW%d" % i], p[g + "_gatas%d" % i],
                 p[g + "_gatad%d" % i], p[g + "_gatb%d" % i])
        x = _leaky(x) + x @ p[g + "_linW%d" % i].T + p[g + "_linb%d" % i]
    x = _gat(x, ei, p[g + "_gatWL"], p[g + "_gatasL"], p[g + "_gatadL"], p[g + "_gatbL"])
    return _sort_pool(x, batch, KTOP)


def _conv1d(x, w, b, stride=1, pad=0):
    y = jax.lax.conv_general_dilated(x, w, (stride,), [(pad, pad)],
                                     dimension_numbers=("NCH", "OIH", "NCH"))
    return y + b[None, :, None]


def _maxpool(x):
    return jax.lax.reduce_window(x, -jnp.inf, jax.lax.max, (1, 1, 2), (1, 1, 2), "VALID")


def _ln(x, g, b, eps=1e-5):
    mu = x.mean(-1, keepdims=True)
    var = ((x - mu) ** 2).mean(-1, keepdims=True)
    return (x - mu) / jnp.sqrt(var + eps) * g + b


def kernel(x_topo, edge_index_topo, x_topo_batch, x_lc, edge_index_lc, x_lc_batch, params):
    xt = _gnn(x_topo, edge_index_topo, x_topo_batch, params, "topo")
    xl = _gnn(x_lc, edge_index_lc, x_lc_batch, params, "lc")
    x = jnp.concatenate([xt, xl], axis=-1)
    x = _ln(x, params["ln_g"], params["ln_b"])
    x = x.reshape(-1, 1, x.shape[-1])
    x = _conv1d(x, params["c1w"], params["c1b"], stride=DOUT)
    x = _leaky(x)
    x = _maxpool(x)
    x = _conv1d(x, params["c2w"], params["c2b"], pad=4)
    x = _leaky(x)
    x = _maxpool(x)
    x = _conv1d(x, params["c3w"], params["c3b"], pad=4)
    x = x.reshape(x.shape[0], -1)
    x = _leaky(x @ params["m1w"].T + params["m1b"])
    x = _leaky(x @ params["m2w"].T + params["m2b"])
    x = x @ params["m3w"].T + params["m3b"]
    return x


# speedup vs baseline: 11.4031x; 10.9138x over previous
import functools

import jax
import jax.numpy as jnp
from jax import lax
from jax.experimental import pallas as pl
from jax.experimental.pallas import tpu as pltpu
from jax.experimental.pallas import tpu_sc as plsc

NB = 50
KTOP = 30
KPAD = 32
HIDS = [128, 128]
DOUT = 64

NPAD = 10240
RPT = NPAD // 16
EAP = 327680
CHA = 2048
CHB = 128
NTILE = 32
NCHA = EAP // (CHA * NTILE)
NCHB = EAP // (CHB * NTILE)


def _leaky(x, s=0.01):
    return jnp.where(x >= 0, x, s * x)


def _lk2(x):
    return jnp.where(x >= 0, x, 0.2 * x)


def _edge_a_body(src_h, dst_h, al_h, ad_h, mx_h, zn_h, ex_h, den_h,
                 srcb, dstb, exb, al_v, ad_v, mx_v, den_v):
    c = lax.axis_index("c")
    s = lax.axis_index("s")
    wid = s * 2 + c
    pltpu.sync_copy(al_h, al_v)
    pltpu.sync_copy(ad_h, ad_v)
    pltpu.sync_copy(mx_h, mx_v)
    pltpu.sync_copy(zn_h, den_v)
    mx = mx_v[...]

    for ci in range(NCHA):
        base = (wid * NCHA + ci) * CHA
        pltpu.sync_copy(src_h.at[pl.ds(base, CHA)], srcb)
        pltpu.sync_copy(dst_h.at[pl.ds(base, CHA)], dstb)

        def inner(j, carry):
            sl = pl.ds(j * 16, 16)
            sv = srcb[sl]
            dv = dstb[sl]
            a1 = plsc.load_gather(al_v, [sv])
            a2 = plsc.load_gather(ad_v, [dv])
            t = a1 + a2
            t = jnp.where(t >= 0, t, 0.2 * t)
            mh = mx + a2
            mh = jnp.where(mh >= 0, mh, 0.2 * mh)
            ex = jnp.exp(t - mh)
            exb[sl] = ex
            plsc.addupdate_scatter(den_v, [dv], ex)
            return carry

        lax.fori_loop(0, CHA // 16, inner, 0)
        pltpu.sync_copy(exb, ex_h.at[pl.ds(base, CHA)])

    pltpu.sync_copy(den_v, den_h.at[wid])


@functools.lru_cache(maxsize=None)
def _edge_a_call():
    mesh = plsc.VectorSubcoreMesh(core_axis_name="c", subcore_axis_name="s")
    return pl.kernel(
        _edge_a_body,
        mesh=mesh,
        compiler_params=pltpu.CompilerParams(needs_layout_passes=False),
        out_type=[
            jax.ShapeDtypeStruct((EAP,), jnp.float32),
            jax.ShapeDtypeStruct((NTILE, NPAD), jnp.float32),
        ],
        scratch_types=[
            pltpu.VMEM((CHA,), jnp.int32),
            pltpu.VMEM((CHA,), jnp.int32),
            pltpu.VMEM((CHA,), jnp.float32),
            pltpu.VMEM((NPAD,), jnp.float32),
            pltpu.VMEM((NPAD,), jnp.float32),
            pltpu.VMEM((16,), jnp.float32),
            pltpu.VMEM((NPAD,), jnp.float32),
        ],
    )


def _edge_b_body(hdim, src_h, dst_h, ex_h, inv_h, xl_h, zc_h, outp_h,
                 srcb, dstb, exb, alb, inv_v, rows_v, out_s, sem):
    c = lax.axis_index("c")
    s = lax.axis_index("s")
    wid = s * 2 + c
    pltpu.sync_copy(inv_h, inv_v)
    pltpu.sync_copy(zc_h.at[pl.ds(s * RPT, RPT)], out_s.at[pl.ds(s * RPT, RPT)])
    plsc.subcore_barrier()

    for ci in range(NCHB):
        base = (wid * NCHB + ci) * CHB
        pltpu.sync_copy(src_h.at[pl.ds(base, CHB)], srcb)
        pltpu.sync_copy(dst_h.at[pl.ds(base, CHB)], dstb)
        pltpu.sync_copy(ex_h.at[pl.ds(base, CHB)], exb)
        pltpu.async_copy(xl_h.at[srcb], rows_v, sem).wait()

        def alphloop(j, carry):
            sl = pl.ds(j * 16, 16)
            dv = dstb[sl]
            iv = plsc.load_gather(inv_v, [dv])
            alb[sl] = exb[sl] * iv
            return carry

        lax.fori_loop(0, CHB // 16, alphloop, 0)

        def scale(e, carry):
            ab = plsc.load_gather(alb, [jnp.broadcast_to(e, (16,))])
            for h in range(hdim // 16):
                sl = pl.ds(h * 16, 16)
                rows_v[e, sl] = rows_v[e, sl] * ab
            return carry

        lax.fori_loop(0, CHB, scale, 0)
        pltpu.sync_copy(rows_v, out_s.at[dstb], add=True)

    plsc.subcore_barrier()
    pltpu.sync_copy(out_s.at[pl.ds(s * RPT, RPT)], outp_h.at[c, pl.ds(s * RPT, RPT)])


@functools.lru_cache(maxsize=None)
def _edge_b_call(hdim):
    mesh = plsc.VectorSubcoreMesh(core_axis_name="c", subcore_axis_name="s")
    return pl.kernel(
        functools.partial(_edge_b_body, hdim),
        mesh=mesh,
        compiler_params=pltpu.CompilerParams(needs_layout_passes=False),
        out_type=[
            jax.ShapeDtypeStruct((2, NPAD, hdim), jnp.float32),
        ],
        scratch_types=[
            pltpu.VMEM((CHB,), jnp.int32),
            pltpu.VMEM((CHB,), jnp.int32),
            pltpu.VMEM((CHB,), jnp.float32),
            pltpu.VMEM((CHB,), jnp.float32),
            pltpu.VMEM((NPAD,), jnp.float32),
            pltpu.VMEM((CHB, hdim), jnp.float32),
            pltpu.VMEM_SHARED((NPAD, hdim), jnp.float32),
            pltpu.SemaphoreType.DMA,
        ],
    )


def _prep_edges(ei, n):
    src0 = ei[0].astype(jnp.int32)
    dst0 = ei[1].astype(jnp.int32)
    e = src0.shape[0]
    dst_eff = jnp.where(src0 == dst0, jnp.int32(n), dst0)
    src = jnp.full((EAP,), n, jnp.int32).at[:e].set(src0)
    dst = jnp.full((EAP,), n, jnp.int32).at[:e].set(dst_eff)
    return src, dst


def _gat(x, prep, W, a_s, a_d, bb):
    src, dst = prep
    n = x.shape[0]
    xl = x @ W
    h = xl.shape[1]
    al = (xl * a_s).sum(-1)
    ad = (xl * a_d).sum(-1)
    maxal = jnp.max(al)
    al_p = jnp.zeros((NPAD,), jnp.float32).at[:n].set(al)
    ad_p = jnp.zeros((NPAD,), jnp.float32).at[:n].set(ad)
    mx = jnp.full((16,), maxal, jnp.float32)
    zn = jnp.zeros((NPAD,), jnp.float32)

    ex, den_parts = _edge_a_call()(src, dst, al_p, ad_p, mx, zn)
    den_e = den_parts.sum(0)[:n]

    ex_self = jnp.exp(_lk2(al + ad) - _lk2(maxal + ad))
    inv = 1.0 / (den_e + ex_self + 1e-16)
    inv_p = jnp.zeros((NPAD,), jnp.float32).at[:n].set(inv)
    hp = 128
    xl_p = jnp.zeros((NPAD, hp), jnp.float32).at[:n, :h].set(xl)
    zc = jnp.zeros((NPAD, hp), jnp.float32)

    (outp,) = _edge_b_call(hp)(src, dst, ex, inv_p, xl_p, zc)
    return outp[0, :n, :h] + outp[1, :n, :h] + (ex_self * inv)[:, None] * xl + bb


def _topk_body(keys_ref, batch_ref, idx_ref, ok_ref, valid_ref):
    b = pl.program_id(0)
    keys = keys_ref[...]
    bat = batch_ref[...]
    rows = jax.lax.broadcasted_iota(jnp.int32, keys.shape, 0)
    lanes = jax.lax.broadcasted_iota(jnp.int32, keys.shape, 1)
    lin = rows * 128 + lanes
    neg_inf = jnp.float32(-jnp.inf)
    big = jnp.int32(2**30)
    kiota = jax.lax.broadcasted_iota(jnp.int32, (1, KPAD), 1)

    valid_ref[...] = (bat == b).astype(jnp.int32)
    idx_ref[0, :, :] = jnp.zeros((1, KPAD), jnp.int32)
    ok_ref[0, :, :] = jnp.zeros((1, KPAD), jnp.int32)

    def body(k, carry):
        valid = valid_ref[...] != 0
        mk = jnp.where(valid, keys, neg_inf)
        m = jnp.max(mk)
        has = m > neg_inf
        cand = jnp.where(valid & (keys == m), lin, big)
        idx = jnp.min(cand)
        sel = (kiota == k) & has
        idx_ref[0, :, :] = jnp.where(sel, idx, idx_ref[0, :, :])
        ok_ref[0, :, :] = jnp.where(sel, 1, ok_ref[0, :, :])
        valid_ref[...] = jnp.where(lin != idx, valid_ref[...], 0)
        return carry

    jax.lax.fori_loop(0, KTOP, body, 0)


def _sort_pool(x, batch, k):
    n, c = x.shape
    npd = ((n + 1023) // 1024) * 1024
    keys = x[:, -1]
    keys_p = jnp.full((npd,), -jnp.inf, jnp.float32).at[:n].set(keys)
    batch_p = jnp.full((npd,), -1, jnp.int32).at[:n].set(batch.astype(jnp.int32))
    r = npd // 128
    keys2 = keys_p.reshape(r, 128)
    batch2 = batch_p.reshape(r, 128)

    idx, ok = pl.pallas_call(
        _topk_body,
        grid=(NB,),
        in_specs=[
            pl.BlockSpec((r, 128), lambda b: (0, 0)),
            pl.BlockSpec((r, 128), lambda b: (0, 0)),
        ],
        out_specs=[
            pl.BlockSpec((1, 1, KPAD), lambda b: (b, 0, 0)),
            pl.BlockSpec((1, 1, KPAD), lambda b: (b, 0, 0)),
        ],
        out_shape=[
            jax.ShapeDtypeStruct((NB, 1, KPAD), jnp.int32),
            jax.ShapeDtypeStruct((NB, 1, KPAD), jnp.int32),
        ],
        scratch_shapes=[pltpu.VMEM((r, 128), jnp.int32)],
    )(keys2, batch2)

    idx = idx.reshape(NB, KPAD)[:, :k].reshape(-1)
    ok = ok.reshape(NB, KPAD)[:, :k].reshape(-1)
    rows = x[idx] * ok[:, None].astype(x.dtype)
    return rows.reshape(NB, k * c)


def _gnn(x, ei, batch, p, g):
    prep = _prep_edges(ei, x.shape[0])
    for i in range(len(HIDS)):
        x = _gat(x, prep, p[g + "_gatW%d" % i], p[g + "_gatas%d" % i],
                 p[g + "_gatad%d" % i], p[g + "_gatb%d" % i])
        x = _leaky(x) + x @ p[g + "_linW%d" % i].T + p[g + "_linb%d" % i]
    x = _gat(x, prep, p[g + "_gatWL"], p[g + "_gatasL"], p[g + "_gatadL"], p[g + "_gatbL"])
    return _sort_pool(x, batch, KTOP)


def _conv1d(x, w, b, stride=1, pad=0):
    y = jax.lax.conv_general_dilated(x, w, (stride,), [(pad, pad)],
                                     dimension_numbers=("NCH", "OIH", "NCH"))
    return y + b[None, :, None]


def _maxpool(x):
    return jax.lax.reduce_window(x, -jnp.inf, jax.lax.max, (1, 1, 2), (1, 1, 2), "VALID")


def _ln(x, g, b, eps=1e-5):
    mu = x.mean(-1, keepdims=True)
    var = ((x - mu) ** 2).mean(-1, keepdims=True)
    return (x - mu) / jnp.sqrt(var + eps) * g + b


def kernel(x_topo, edge_index_topo, x_topo_batch, x_lc, edge_index_lc, x_lc_batch, params):
    xt = _gnn(x_topo, edge_index_topo, x_topo_batch, params, "topo")
    xl = _gnn(x_lc, edge_index_lc, x_lc_batch, params, "lc")
    x = jnp.concatenate([xt, xl], axis=-1)
    x = _ln(x, params["ln_g"], params["ln_b"])
    x = x.reshape(-1, 1, x.shape[-1])
    x = _conv1d(x, params["c1w"], params["c1b"], stride=DOUT)
    x = _leaky(x)
    x = _maxpool(x)
    x = _conv1d(x, params["c2w"], params["c2b"], pad=4)
    x = _leaky(x)
    x = _maxpool(x)
    x = _conv1d(x, params["c3w"], params["c3b"], pad=4)
    x = x.reshape(x.shape[0], -1)
    x = _leaky(x @ params["m1w"].T + params["m1b"])
    x = _leaky(x @ params["m2w"].T + params["m2b"])
    x = x @ params["m3w"].T + params["m3b"]
    return x


# speedup vs baseline: 14.8857x; 1.3054x over previous
import functools

import jax
import jax.numpy as jnp
from jax import lax
from jax.experimental import pallas as pl
from jax.experimental.pallas import tpu as pltpu
from jax.experimental.pallas import tpu_sc as plsc

NB = 50
KTOP = 30
KPAD = 32
HIDS = [128, 128]
DOUT = 64

NPAD = 10240
RPT = NPAD // 16
EAP = 327680
CHA = 2048
CHB = 128
KB = CHB // 128
HP = 128
NTILE = 32
NCHA = EAP // (CHA * NTILE)
NCHB = EAP // (CHB * NTILE)


def _leaky(x, s=0.01):
    return jnp.where(x >= 0, x, s * x)


def _lk2(x):
    return jnp.where(x >= 0, x, 0.2 * x)


def _edge_a_body(src_h, dst_h, al_h, ad_h, mx_h, zn_h, ex_h, den_h,
                 srcb0, srcb1, srcb2, dstb0, dstb1, dstb2, exb0, exb1,
                 al_v, ad_v, mx_v, den_v, semi0, semi1, semi2, semw0, semw1):
    c = lax.axis_index("c")
    s = lax.axis_index("s")
    wid = s * 2 + c
    srcbs = (srcb0, srcb1, srcb2)
    dstbs = (dstb0, dstb1, dstb2)
    exbs = (exb0, exb1)
    semis = (semi0, semi1, semi2)
    semws = (semw0, semw1)
    pltpu.sync_copy(al_h, al_v)
    pltpu.sync_copy(ad_h, ad_v)
    pltpu.sync_copy(mx_h, mx_v)
    pltpu.sync_copy(zn_h, den_v)
    mx = mx_v[...]

    def issue_idx(ci):
        q = ci % 3
        base = (wid * NCHA + ci) * CHA
        h1 = pltpu.async_copy(src_h.at[pl.ds(base, CHA)], srcbs[q], semis[q])
        h2 = pltpu.async_copy(dst_h.at[pl.ds(base, CHA)], dstbs[q], semis[q])
        return (h1, h2)

    idx_h = {0: issue_idx(0), 1: issue_idx(1)}
    wb_h = {}
    for ci in range(NCHA):
        q = ci % 3
        p = ci % 2
        if ci >= 2:
            wb_h[p].wait()
        for h in idx_h[q]:
            h.wait()
        if ci + 2 < NCHA:
            idx_h[(ci + 2) % 3] = issue_idx(ci + 2)
        srcb = srcbs[q]
        dstb = dstbs[q]
        exb = exbs[p]

        def inner(j, carry):
            sl = pl.ds(j * 16, 16)
            sv = srcb[sl]
            dv = dstb[sl]
            a1 = plsc.load_gather(al_v, [sv])
            a2 = plsc.load_gather(ad_v, [dv])
            t = a1 + a2
            t = jnp.where(t >= 0, t, 0.2 * t)
            mh = mx + a2
            mh = jnp.where(mh >= 0, mh, 0.2 * mh)
            ex = jnp.exp(t - mh)
            exb[sl] = ex
            plsc.addupdate_scatter(den_v, [dv], ex)
            return carry

        lax.fori_loop(0, CHA // 16, inner, 0, unroll=4)
        base = (wid * NCHA + ci) * CHA
        wb_h[p] = pltpu.async_copy(exb, ex_h.at[pl.ds(base, CHA)], semws[p])

    wb_h[0].wait()
    wb_h[1].wait()
    pltpu.sync_copy(den_v, den_h.at[wid])


@functools.lru_cache(maxsize=None)
def _edge_a_call():
    mesh = plsc.VectorSubcoreMesh(core_axis_name="c", subcore_axis_name="s")
    return pl.kernel(
        _edge_a_body,
        mesh=mesh,
        compiler_params=pltpu.CompilerParams(needs_layout_passes=False),
        out_type=[
            jax.ShapeDtypeStruct((EAP,), jnp.float32),
            jax.ShapeDtypeStruct((NTILE, NPAD), jnp.float32),
        ],
        scratch_types=[
            pltpu.VMEM((CHA,), jnp.int32),
            pltpu.VMEM((CHA,), jnp.int32),
            pltpu.VMEM((CHA,), jnp.int32),
            pltpu.VMEM((CHA,), jnp.int32),
            pltpu.VMEM((CHA,), jnp.int32),
            pltpu.VMEM((CHA,), jnp.int32),
            pltpu.VMEM((CHA,), jnp.float32),
            pltpu.VMEM((CHA,), jnp.float32),
            pltpu.VMEM((NPAD,), jnp.float32),
            pltpu.VMEM((NPAD,), jnp.float32),
            pltpu.VMEM((16,), jnp.float32),
            pltpu.VMEM((NPAD,), jnp.float32),
            pltpu.SemaphoreType.DMA,
            pltpu.SemaphoreType.DMA,
            pltpu.SemaphoreType.DMA,
            pltpu.SemaphoreType.DMA,
            pltpu.SemaphoreType.DMA,
        ],
    )


def _edge_b_body(src_h, dst_h, ex_h, inv_h, xl_h, zc_h, outp_h,
                 srcb0, srcb1, dstb0, dstb1, exb0, exb1,
                 alb, inv_v, rows0, rows1, out_s,
                 semi0, semi1, semr0, semr1, sems0, sems1):
    c = lax.axis_index("c")
    s = lax.axis_index("s")
    wid = s * 2 + c
    srcbs = (srcb0, srcb1)
    dstbs = (dstb0, dstb1)
    exbs = (exb0, exb1)
    semis = (semi0, semi1)
    rows = (rows0, rows1)
    semrs = (semr0, semr1)
    semss = (sems0, sems1)
    pltpu.sync_copy(inv_h, inv_v)
    pltpu.sync_copy(zc_h.at[pl.ds(s * RPT, RPT)], out_s.at[pl.ds(s * RPT, RPT)])
    plsc.subcore_barrier()

    cbase = wid * NCHB * CHB

    def issue_idx(base, p):
        return [
            pltpu.async_copy(src_h.at[pl.ds(base, CHB)], srcbs[p], semis[p]),
            pltpu.async_copy(dst_h.at[pl.ds(base, CHB)], dstbs[p], semis[p]),
            pltpu.async_copy(ex_h.at[pl.ds(base, CHB)], exbs[p], semis[p]),
        ]

    def wait_idx(p):
        pltpu.make_async_copy(src_h.at[pl.ds(0, CHB)], srcbs[p], semis[p]).wait()
        pltpu.make_async_copy(dst_h.at[pl.ds(0, CHB)], dstbs[p], semis[p]).wait()
        pltpu.make_async_copy(ex_h.at[pl.ds(0, CHB)], exbs[p], semis[p]).wait()

    def issue_gather(p):
        return pltpu.async_copy(xl_h.at[srcbs[p]], rows[p], semrs[p])

    def issue_scatter(p):
        return pltpu.async_copy(rows[p], out_s.at[dstbs[p]], semss[p], add=True)

    def wait_scatter(p):
        pltpu.make_async_copy(rows[p], out_s.at[dstbs[p]], semss[p]).wait()

    def compute(p):
        exb = exbs[p]
        dstb = dstbs[p]
        rowsp = rows[p]

        def alphloop(j, carry):
            sl = pl.ds(j * 16, 16)
            iv = plsc.load_gather(inv_v, [dstb[sl]])
            alb[sl] = exb[sl] * iv
            return carry

        lax.fori_loop(0, CHB // 16, alphloop, 0, unroll=4)

        def scale(e, carry):
            ab = plsc.load_gather(alb, [jnp.broadcast_to(e, (16,))])
            for hh in range(HP // 16):
                sl = pl.ds(hh * 16, 16)
                rowsp[e, sl] = rowsp[e, sl] * ab
            return carry

        lax.fori_loop(0, CHB, scale, 0, unroll=2)

    h = issue_idx(cbase, 0)
    for x in h:
        x.wait()
    g0 = issue_gather(0)
    h = issue_idx(cbase + CHB, 1)
    for x in h:
        x.wait()
    g1 = issue_gather(1)
    g0.wait()
    compute(0)
    issue_scatter(0)
    g1.wait()
    compute(1)
    issue_scatter(1)
    wait_scatter(0)
    issue_idx(cbase + 2 * CHB, 0)

    def pair(a, last):
        wait_idx(0)
        ga = issue_gather(0)
        wait_scatter(1)
        issue_idx(a + CHB, 1)
        wait_idx(1)
        gb = issue_gather(1)
        ga.wait()
        compute(0)
        issue_scatter(0)
        gb.wait()
        compute(1)
        issue_scatter(1)
        wait_scatter(0)
        if not last:
            issue_idx(a + 2 * CHB, 0)

    def body(t, carry):
        pair(cbase + (2 * t) * CHB, False)
        return carry

    lax.fori_loop(1, NCHB // 2 - 1, body, 0)
    pair(cbase + (NCHB - 2) * CHB, True)
    wait_scatter(1)
    plsc.subcore_barrier()
    pltpu.sync_copy(out_s.at[pl.ds(s * RPT, RPT)], outp_h.at[c, pl.ds(s * RPT, RPT)])


@functools.lru_cache(maxsize=None)
def _edge_b_call():
    mesh = plsc.VectorSubcoreMesh(core_axis_name="c", subcore_axis_name="s")
    return pl.kernel(
        _edge_b_body,
        mesh=mesh,
        compiler_params=pltpu.CompilerParams(needs_layout_passes=False),
        out_type=[
            jax.ShapeDtypeStruct((2, NPAD, HP), jnp.float32),
        ],
        scratch_types=[
            pltpu.VMEM((CHB,), jnp.int32),
            pltpu.VMEM((CHB,), jnp.int32),
            pltpu.VMEM((CHB,), jnp.int32),
            pltpu.VMEM((CHB,), jnp.int32),
            pltpu.VMEM((CHB,), jnp.float32),
            pltpu.VMEM((CHB,), jnp.float32),
            pltpu.VMEM((CHB,), jnp.float32),
            pltpu.VMEM((NPAD,), jnp.float32),
            pltpu.VMEM((CHB, HP), jnp.float32),
            pltpu.VMEM((CHB, HP), jnp.float32),
            pltpu.VMEM_SHARED((NPAD, HP), jnp.float32),
            pltpu.SemaphoreType.DMA,
            pltpu.SemaphoreType.DMA,
            pltpu.SemaphoreType.DMA,
            pltpu.SemaphoreType.DMA,
            pltpu.SemaphoreType.DMA,
            pltpu.SemaphoreType.DMA,
        ],
    )


def _prep_edges(ei, n):
    src0 = ei[0].astype(jnp.int32)
    dst0 = ei[1].astype(jnp.int32)
    e = src0.shape[0]
    dst_eff = jnp.where(src0 == dst0, jnp.int32(n), dst0)
    src = jnp.full((EAP,), n, jnp.int32).at[:e].set(src0)
    dst = jnp.full((EAP,), n, jnp.int32).at[:e].set(dst_eff)
    return src, dst


def _gat(x, prep, W, a_s, a_d, bb):
    src, dst = prep
    n = x.shape[0]
    xl = x @ W
    h = xl.shape[1]
    al = (xl * a_s).sum(-1)
    ad = (xl * a_d).sum(-1)
    maxal = jnp.max(al)
    al_p = jnp.zeros((NPAD,), jnp.float32).at[:n].set(al)
    ad_p = jnp.zeros((NPAD,), jnp.float32).at[:n].set(ad)
    mx = jnp.full((16,), maxal, jnp.float32)
    zn = jnp.zeros((NPAD,), jnp.float32)

    ex, den_parts = _edge_a_call()(src, dst, al_p, ad_p, mx, zn)
    den_e = den_parts.sum(0)[:n]

    ex_self = jnp.exp(_lk2(al + ad) - _lk2(maxal + ad))
    inv = 1.0 / (den_e + ex_self + 1e-16)
    inv_p = jnp.zeros((NPAD,), jnp.float32).at[:n].set(inv)
    xl_p = jnp.zeros((NPAD, HP), jnp.float32).at[:n, :h].set(xl)
    zc = jnp.zeros((NPAD, HP), jnp.float32)

    (outp,) = _edge_b_call()(src, dst, ex, inv_p, xl_p, zc)
    return outp[0, :n, :h] + outp[1, :n, :h] + (ex_self * inv)[:, None] * xl + bb


def _topk_body(keys_ref, batch_ref, idx_ref, ok_ref, valid_ref):
    b = pl.program_id(0)
    keys = keys_ref[...]
    bat = batch_ref[...]
    rows = jax.lax.broadcasted_iota(jnp.int32, keys.shape, 0)
    lanes = jax.lax.broadcasted_iota(jnp.int32, keys.shape, 1)
    lin = rows * 128 + lanes
    neg_inf = jnp.float32(-jnp.inf)
    big = jnp.int32(2**30)
    kiota = jax.lax.broadcasted_iota(jnp.int32, (1, KPAD), 1)

    valid_ref[...] = (bat == b).astype(jnp.int32)
    idx_ref[0, :, :] = jnp.zeros((1, KPAD), jnp.int32)
    ok_ref[0, :, :] = jnp.zeros((1, KPAD), jnp.int32)

    def body(k, carry):
        valid = valid_ref[...] != 0
        mk = jnp.where(valid, keys, neg_inf)
        m = jnp.max(mk)
        has = m > neg_inf
        cand = jnp.where(valid & (keys == m), lin, big)
        idx = jnp.min(cand)
        sel = (kiota == k) & has
        idx_ref[0, :, :] = jnp.where(sel, idx, idx_ref[0, :, :])
        ok_ref[0, :, :] = jnp.where(sel, 1, ok_ref[0, :, :])
        valid_ref[...] = jnp.where(lin != idx, valid_ref[...], 0)
        return carry

    jax.lax.fori_loop(0, KTOP, body, 0)


def _sort_pool(x, batch, k):
    n, c = x.shape
    npd = ((n + 1023) // 1024) * 1024
    keys = x[:, -1]
    keys_p = jnp.full((npd,), -jnp.inf, jnp.float32).at[:n].set(keys)
    batch_p = jnp.full((npd,), -1, jnp.int32).at[:n].set(batch.astype(jnp.int32))
    r = npd // 128
    keys2 = keys_p.reshape(r, 128)
    batch2 = batch_p.reshape(r, 128)

    idx, ok = pl.pallas_call(
        _topk_body,
        grid=(NB,),
        in_specs=[
            pl.BlockSpec((r, 128), lambda b: (0, 0)),
            pl.BlockSpec((r, 128), lambda b: (0, 0)),
        ],
        out_specs=[
            pl.BlockSpec((1, 1, KPAD), lambda b: (b, 0, 0)),
            pl.BlockSpec((1, 1, KPAD), lambda b: (b, 0, 0)),
        ],
        out_shape=[
            jax.ShapeDtypeStruct((NB, 1, KPAD), jnp.int32),
            jax.ShapeDtypeStruct((NB, 1, KPAD), jnp.int32),
        ],
        scratch_shapes=[pltpu.VMEM((r, 128), jnp.int32)],
    )(keys2, batch2)

    idx = idx.reshape(NB, KPAD)[:, :k].reshape(-1)
    ok = ok.reshape(NB, KPAD)[:, :k].reshape(-1)
    rows = x[idx] * ok[:, None].astype(x.dtype)
    return rows.reshape(NB, k * c)


def _gnn(x, ei, batch, p, g):
    prep = _prep_edges(ei, x.shape[0])
    for i in range(len(HIDS)):
        x = _gat(x, prep, p[g + "_gatW%d" % i], p[g + "_gatas%d" % i],
                 p[g + "_gatad%d" % i], p[g + "_gatb%d" % i])
        x = _leaky(x) + x @ p[g + "_linW%d" % i].T + p[g + "_linb%d" % i]
    x = _gat(x, prep, p[g + "_gatWL"], p[g + "_gatasL"], p[g + "_gatadL"], p[g + "_gatbL"])
    return _sort_pool(x, batch, KTOP)


def _conv1d(x, w, b, stride=1, pad=0):
    y = jax.lax.conv_general_dilated(x, w, (stride,), [(pad, pad)],
                                     dimension_numbers=("NCH", "OIH", "NCH"))
    return y + b[None, :, None]


def _maxpool(x):
    return jax.lax.reduce_window(x, -jnp.inf, jax.lax.max, (1, 1, 2), (1, 1, 2), "VALID")


def _ln(x, g, b, eps=1e-5):
    mu = x.mean(-1, keepdims=True)
    var = ((x - mu) ** 2).mean(-1, keepdims=True)
    return (x - mu) / jnp.sqrt(var + eps) * g + b


def kernel(x_topo, edge_index_topo, x_topo_batch, x_lc, edge_index_lc, x_lc_batch, params):
    xt = _gnn(x_topo, edge_index_topo, x_topo_batch, params, "topo")
    xl = _gnn(x_lc, edge_index_lc, x_lc_batch, params, "lc")
    x = jnp.concatenate([xt, xl], axis=-1)
    x = _ln(x, params["ln_g"], params["ln_b"])
    x = x.reshape(-1, 1, x.shape[-1])
    x = _conv1d(x, params["c1w"], params["c1b"], stride=DOUT)
    x = _leaky(x)
    x = _maxpool(x)
    x = _conv1d(x, params["c2w"], params["c2b"], pad=4)
    x = _leaky(x)
    x = _maxpool(x)
    x = _conv1d(x, params["c3w"], params["c3b"], pad=4)
    x = x.reshape(x.shape[0], -1)
    x = _leaky(x @ params["m1w"].T + params["m1b"])
    x = _leaky(x @ params["m2w"].T + params["m2b"])
    x = x @ params["m3w"].T + params["m3b"]
    return x


# speedup vs baseline: 15.2862x; 1.0269x over previous
import functools

import jax
import jax.numpy as jnp
from jax import lax
from jax.experimental import pallas as pl
from jax.experimental.pallas import tpu as pltpu
from jax.experimental.pallas import tpu_sc as plsc

NB = 50
KTOP = 30
KPAD = 32
HIDS = [128, 128]
DOUT = 64

NPAD = 10240
RPT = NPAD // 16
EAP = 327680
CHA = 2048
CHB = 128
KB = CHB // 128
HP = 128
NTILE = 32
NCHA = EAP // (CHA * NTILE)
NCHB = EAP // (CHB * NTILE)


def _leaky(x, s=0.01):
    return jnp.where(x >= 0, x, s * x)


def _lk2(x):
    return jnp.where(x >= 0, x, 0.2 * x)


def _edge_a_body(src_h, dst_h, al_h, ad_h, mx_h, zn_h, ex_h, den_h,
                 srcb0, srcb1, srcb2, dstb0, dstb1, dstb2, exb0, exb1,
                 al_v, ad_v, mx_v, den_v, semi0, semi1, semi2, semw0, semw1):
    c = lax.axis_index("c")
    s = lax.axis_index("s")
    wid = s * 2 + c
    srcbs = (srcb0, srcb1, srcb2)
    dstbs = (dstb0, dstb1, dstb2)
    exbs = (exb0, exb1)
    semis = (semi0, semi1, semi2)
    semws = (semw0, semw1)
    pltpu.sync_copy(al_h, al_v)
    pltpu.sync_copy(ad_h, ad_v)
    pltpu.sync_copy(mx_h, mx_v)
    pltpu.sync_copy(zn_h, den_v)
    mx = mx_v[...]

    def issue_idx(ci):
        q = ci % 3
        base = (wid * NCHA + ci) * CHA
        h1 = pltpu.async_copy(src_h.at[pl.ds(base, CHA)], srcbs[q], semis[q])
        h2 = pltpu.async_copy(dst_h.at[pl.ds(base, CHA)], dstbs[q], semis[q])
        return (h1, h2)

    idx_h = {0: issue_idx(0), 1: issue_idx(1)}
    wb_h = {}
    for ci in range(NCHA):
        q = ci % 3
        p = ci % 2
        if ci >= 2:
            wb_h[p].wait()
        for h in idx_h[q]:
            h.wait()
        if ci + 2 < NCHA:
            idx_h[(ci + 2) % 3] = issue_idx(ci + 2)
        srcb = srcbs[q]
        dstb = dstbs[q]
        exb = exbs[p]

        def inner(j, carry):
            sl = pl.ds(j * 16, 16)
            sv = srcb[sl]
            dv = dstb[sl]
            a1 = plsc.load_gather(al_v, [sv])
            a2 = plsc.load_gather(ad_v, [dv])
            t = a1 + a2
            t = jnp.where(t >= 0, t, 0.2 * t)
            mh = mx + a2
            mh = jnp.where(mh >= 0, mh, 0.2 * mh)
            ex = jnp.exp(t - mh)
            exb[sl] = ex
            plsc.addupdate_scatter(den_v, [dv], ex)
            return carry

        lax.fori_loop(0, CHA // 16, inner, 0, unroll=4)
        base = (wid * NCHA + ci) * CHA
        wb_h[p] = pltpu.async_copy(exb, ex_h.at[pl.ds(base, CHA)], semws[p])

    wb_h[0].wait()
    wb_h[1].wait()
    pltpu.sync_copy(den_v, den_h.at[wid])


@functools.lru_cache(maxsize=None)
def _edge_a_call():
    mesh = plsc.VectorSubcoreMesh(core_axis_name="c", subcore_axis_name="s")
    return pl.kernel(
        _edge_a_body,
        mesh=mesh,
        compiler_params=pltpu.CompilerParams(needs_layout_passes=False),
        out_type=[
            jax.ShapeDtypeStruct((EAP,), jnp.float32),
            jax.ShapeDtypeStruct((NTILE, NPAD), jnp.float32),
        ],
        scratch_types=[
            pltpu.VMEM((CHA,), jnp.int32),
            pltpu.VMEM((CHA,), jnp.int32),
            pltpu.VMEM((CHA,), jnp.int32),
            pltpu.VMEM((CHA,), jnp.int32),
            pltpu.VMEM((CHA,), jnp.int32),
            pltpu.VMEM((CHA,), jnp.int32),
            pltpu.VMEM((CHA,), jnp.float32),
            pltpu.VMEM((CHA,), jnp.float32),
            pltpu.VMEM((NPAD,), jnp.float32),
            pltpu.VMEM((NPAD,), jnp.float32),
            pltpu.VMEM((16,), jnp.float32),
            pltpu.VMEM((NPAD,), jnp.float32),
            pltpu.SemaphoreType.DMA,
            pltpu.SemaphoreType.DMA,
            pltpu.SemaphoreType.DMA,
            pltpu.SemaphoreType.DMA,
            pltpu.SemaphoreType.DMA,
        ],
    )


def _edge_b_body(src_h, dst_h, ex_h, inv_h, xl_h, zc_h, outp_h,
                 srcb0, srcb1, dstb0, dstb1, exb0, exb1,
                 alb, inv_v, rows0, rows1, out_s,
                 semi0, semi1, semr0, semr1, sems0, sems1):
    c = lax.axis_index("c")
    s = lax.axis_index("s")
    wid = s * 2 + c
    srcbs = (srcb0, srcb1)
    dstbs = (dstb0, dstb1)
    exbs = (exb0, exb1)
    semis = (semi0, semi1)
    rows = (rows0, rows1)
    semrs = (semr0, semr1)
    semss = (sems0, sems1)
    pltpu.sync_copy(inv_h, inv_v)
    pltpu.sync_copy(zc_h.at[pl.ds(s * RPT, RPT)], out_s.at[pl.ds(s * RPT, RPT)])
    plsc.subcore_barrier()

    cbase = wid * NCHB * CHB

    def issue_idx(base, p):
        return [
            pltpu.async_copy(src_h.at[pl.ds(base, CHB)], srcbs[p], semis[p]),
            pltpu.async_copy(dst_h.at[pl.ds(base, CHB)], dstbs[p], semis[p]),
            pltpu.async_copy(ex_h.at[pl.ds(base, CHB)], exbs[p], semis[p]),
        ]

    def wait_idx(p):
        pltpu.make_async_copy(src_h.at[pl.ds(0, CHB)], srcbs[p], semis[p]).wait()
        pltpu.make_async_copy(dst_h.at[pl.ds(0, CHB)], dstbs[p], semis[p]).wait()
        pltpu.make_async_copy(ex_h.at[pl.ds(0, CHB)], exbs[p], semis[p]).wait()

    def issue_gather(p):
        return pltpu.async_copy(xl_h.at[srcbs[p]], rows[p], semrs[p])

    def issue_scatter(p):
        return pltpu.async_copy(rows[p], out_s.at[dstbs[p]], semss[p], add=True)

    def wait_scatter(p):
        pltpu.make_async_copy(rows[p], out_s.at[dstbs[p]], semss[p]).wait()

    def compute(p):
        exb = exbs[p]
        dstb = dstbs[p]
        rowsp = rows[p]

        @plsc.parallel_loop(0, CHB // 16, unroll=2)
        def alphloop(j):
            sl = pl.ds(j * 16, 16)
            iv = plsc.load_gather(inv_v, [dstb[sl]])
            alb[sl] = exb[sl] * iv

        @plsc.parallel_loop(0, CHB, unroll=4)
        def scale(e):
            ab = plsc.load_gather(alb, [jnp.broadcast_to(e, (16,))])
            for hh in range(HP // 16):
                sl = pl.ds(hh * 16, 16)
                rowsp[e, sl] = rowsp[e, sl] * ab

    h = issue_idx(cbase, 0)
    for x in h:
        x.wait()
    g0 = issue_gather(0)
    h = issue_idx(cbase + CHB, 1)
    for x in h:
        x.wait()
    g1 = issue_gather(1)
    g0.wait()
    compute(0)
    issue_scatter(0)
    g1.wait()
    compute(1)
    issue_scatter(1)
    wait_scatter(0)
    issue_idx(cbase + 2 * CHB, 0)

    def pair(a, last):
        wait_idx(0)
        ga = issue_gather(0)
        wait_scatter(1)
        issue_idx(a + CHB, 1)
        wait_idx(1)
        gb = issue_gather(1)
        ga.wait()
        compute(0)
        issue_scatter(0)
        gb.wait()
        compute(1)
        issue_scatter(1)
        wait_scatter(0)
        if not last:
            issue_idx(a + 2 * CHB, 0)

    def body(t, carry):
        pair(cbase + (2 * t) * CHB, False)
        return carry

    lax.fori_loop(1, NCHB // 2 - 1, body, 0)
    pair(cbase + (NCHB - 2) * CHB, True)
    wait_scatter(1)
    plsc.subcore_barrier()
    pltpu.sync_copy(out_s.at[pl.ds(s * RPT, RPT)], outp_h.at[c, pl.ds(s * RPT, RPT)])


@functools.lru_cache(maxsize=None)
def _edge_b_call():
    mesh = plsc.VectorSubcoreMesh(core_axis_name="c", subcore_axis_name="s")
    return pl.kernel(
        _edge_b_body,
        mesh=mesh,
        compiler_params=pltpu.CompilerParams(needs_layout_passes=False),
        out_type=[
            jax.ShapeDtypeStruct((2, NPAD, HP), jnp.float32),
        ],
        scratch_types=[
            pltpu.VMEM((CHB,), jnp.int32),
            pltpu.VMEM((CHB,), jnp.int32),
            pltpu.VMEM((CHB,), jnp.int32),
            pltpu.VMEM((CHB,), jnp.int32),
            pltpu.VMEM((CHB,), jnp.float32),
            pltpu.VMEM((CHB,), jnp.float32),
            pltpu.VMEM((CHB,), jnp.float32),
            pltpu.VMEM((NPAD,), jnp.float32),
            pltpu.VMEM((CHB, HP), jnp.float32),
            pltpu.VMEM((CHB, HP), jnp.float32),
            pltpu.VMEM_SHARED((NPAD, HP), jnp.float32),
            pltpu.SemaphoreType.DMA,
            pltpu.SemaphoreType.DMA,
            pltpu.SemaphoreType.DMA,
            pltpu.SemaphoreType.DMA,
            pltpu.SemaphoreType.DMA,
            pltpu.SemaphoreType.DMA,
        ],
    )


def _prep_edges(ei, n):
    src0 = ei[0].astype(jnp.int32)
    dst0 = ei[1].astype(jnp.int32)
    e = src0.shape[0]
    dst_eff = jnp.where(src0 == dst0, jnp.int32(n), dst0)
    src = jnp.full((EAP,), n, jnp.int32).at[:e].set(src0)
    dst = jnp.full((EAP,), n, jnp.int32).at[:e].set(dst_eff)
    return src, dst


def _gat(x, prep, W, a_s, a_d, bb):
    src, dst = prep
    n = x.shape[0]
    xl = x @ W
    h = xl.shape[1]
    al = (xl * a_s).sum(-1)
    ad = (xl * a_d).sum(-1)
    maxal = jnp.max(al)
    al_p = jnp.zeros((NPAD,), jnp.float32).at[:n].set(al)
    ad_p = jnp.zeros((NPAD,), jnp.float32).at[:n].set(ad)
    mx = jnp.full((16,), maxal, jnp.float32)
    zn = jnp.zeros((NPAD,), jnp.float32)

    ex, den_parts = _edge_a_call()(src, dst, al_p, ad_p, mx, zn)
    den_e = den_parts.sum(0)[:n]

    ex_self = jnp.exp(_lk2(al + ad) - _lk2(maxal + ad))
    inv = 1.0 / (den_e + ex_self + 1e-16)
    inv_p = jnp.zeros((NPAD,), jnp.float32).at[:n].set(inv)
    xl_p = jnp.zeros((NPAD, HP), jnp.float32).at[:n, :h].set(xl)
    zc = jnp.zeros((NPAD, HP), jnp.float32)

    (outp,) = _edge_b_call()(src, dst, ex, inv_p, xl_p, zc)
    return outp[0, :n, :h] + outp[1, :n, :h] + (ex_self * inv)[:, None] * xl + bb


def _topk_body(keys_ref, batch_ref, idx_ref, ok_ref, valid_ref):
    b = pl.program_id(0)
    keys = keys_ref[...]
    bat = batch_ref[...]
    rows = jax.lax.broadcasted_iota(jnp.int32, keys.shape, 0)
    lanes = jax.lax.broadcasted_iota(jnp.int32, keys.shape, 1)
    lin = rows * 128 + lanes
    neg_inf = jnp.float32(-jnp.inf)
    big = jnp.int32(2**30)
    kiota = jax.lax.broadcasted_iota(jnp.int32, (1, KPAD), 1)

    valid_ref[...] = (bat == b).astype(jnp.int32)
    idx_ref[0, :, :] = jnp.zeros((1, KPAD), jnp.int32)
    ok_ref[0, :, :] = jnp.zeros((1, KPAD), jnp.int32)

    def body(k, carry):
        valid = valid_ref[...] != 0
        mk = jnp.where(valid, keys, neg_inf)
        m = jnp.max(mk)
        has = m > neg_inf
        cand = jnp.where(valid & (keys == m), lin, big)
        idx = jnp.min(cand)
        sel = (kiota == k) & has
        idx_ref[0, :, :] = jnp.where(sel, idx, idx_ref[0, :, :])
        ok_ref[0, :, :] = jnp.where(sel, 1, ok_ref[0, :, :])
        valid_ref[...] = jnp.where(lin != idx, valid_ref[...], 0)
        return carry

    jax.lax.fori_loop(0, KTOP, body, 0)


def _sort_pool(x, batch, k):
    n, c = x.shape
    npd = ((n + 1023) // 1024) * 1024
    keys = x[:, -1]
    keys_p = jnp.full((npd,), -jnp.inf, jnp.float32).at[:n].set(keys)
    batch_p = jnp.full((npd,), -1, jnp.int32).at[:n].set(batch.astype(jnp.int32))
    r = npd // 128
    keys2 = keys_p.reshape(r, 128)
    batch2 = batch_p.reshape(r, 128)

    idx, ok = pl.pallas_call(
        _topk_body,
        grid=(NB,),
        in_specs=[
            pl.BlockSpec((r, 128), lambda b: (0, 0)),
            pl.BlockSpec((r, 128), lambda b: (0, 0)),
        ],
        out_specs=[
            pl.BlockSpec((1, 1, KPAD), lambda b: (b, 0, 0)),
            pl.BlockSpec((1, 1, KPAD), lambda b: (b, 0, 0)),
        ],
        out_shape=[
            jax.ShapeDtypeStruct((NB, 1, KPAD), jnp.int32),
            jax.ShapeDtypeStruct((NB, 1, KPAD), jnp.int32),
        ],
        scratch_shapes=[pltpu.VMEM((r, 128), jnp.int32)],
    )(keys2, batch2)

    idx = idx.reshape(NB, KPAD)[:, :k].reshape(-1)
    ok = ok.reshape(NB, KPAD)[:, :k].reshape(-1)
    rows = x[idx] * ok[:, None].astype(x.dtype)
    return rows.reshape(NB, k * c)


def _gnn(x, ei, batch, p, g):
    prep = _prep_edges(ei, x.shape[0])
    for i in range(len(HIDS)):
        x = _gat(x, prep, p[g + "_gatW%d" % i], p[g + "_gatas%d" % i],
                 p[g + "_gatad%d" % i], p[g + "_gatb%d" % i])
        x = _leaky(x) + x @ p[g + "_linW%d" % i].T + p[g + "_linb%d" % i]
    x = _gat(x, prep, p[g + "_gatWL"], p[g + "_gatasL"], p[g + "_gatadL"], p[g + "_gatbL"])
    return _sort_pool(x, batch, KTOP)


def _conv1d(x, w, b, stride=1, pad=0):
    y = jax.lax.conv_general_dilated(x, w, (stride,), [(pad, pad)],
                                     dimension_numbers=("NCH", "OIH", "NCH"))
    return y + b[None, :, None]


def _maxpool(x):
    return jax.lax.reduce_window(x, -jnp.inf, jax.lax.max, (1, 1, 2), (1, 1, 2), "VALID")


def _ln(x, g, b, eps=1e-5):
    mu = x.mean(-1, keepdims=True)
    var = ((x - mu) ** 2).mean(-1, keepdims=True)
    return (x - mu) / jnp.sqrt(var + eps) * g + b


def kernel(x_topo, edge_index_topo, x_topo_batch, x_lc, edge_index_lc, x_lc_batch, params):
    xt = _gnn(x_topo, edge_index_topo, x_topo_batch, params, "topo")
    xl = _gnn(x_lc, edge_index_lc, x_lc_batch, params, "lc")
    x = jnp.concatenate([xt, xl], axis=-1)
    x = _ln(x, params["ln_g"], params["ln_b"])
    x = x.reshape(-1, 1, x.shape[-1])
    x = _conv1d(x, params["c1w"], params["c1b"], stride=DOUT)
    x = _leaky(x)
    x = _maxpool(x)
    x = _conv1d(x, params["c2w"], params["c2b"], pad=4)
    x = _leaky(x)
    x = _maxpool(x)
    x = _conv1d(x, params["c3w"], params["c3b"], pad=4)
    x = x.reshape(x.shape[0], -1)
    x = _leaky(x @ params["m1w"].T + params["m1b"])
    x = _leaky(x @ params["m2w"].T + params["m2b"])
    x = x @ params["m3w"].T + params["m3b"]
    return x
